# initial kernel scaffold (unmeasured)
import jax
import jax.numpy as jnp
from jax import lax
from jax.experimental import pallas as pl
from jax.experimental.pallas import tpu as pltpu

N_DEV = 16


def kernel(x, w_mat):
    m, k_per = x.shape
    _, n = w_mat.shape
    m_per = m // N_DEV

    def body(x_ref, w_ref, out_ref, comm_ref, send_sems, recv_sems):
        my = lax.axis_index("i")
        left = lax.rem(my + N_DEV - 1, N_DEV)
        right = lax.rem(my + 1, N_DEV)

        barrier_sem = pltpu.get_barrier_semaphore()
        for nbr in (left, right):
            pl.semaphore_signal(
                barrier_sem, inc=1,
                device_id=(nbr,), device_id_type=pl.DeviceIdType.MESH,
            )
        pl.semaphore_wait(barrier_sem, 2)

        w = w_ref[:, :]

        def chunk_partial(c):
            return jnp.dot(
                x_ref[pl.ds(c * m_per, m_per), :], w,
                preferred_element_type=jnp.float32,
            )

        comm_ref[0, :, :] = chunk_partial(lax.rem(my + N_DEV - 1, N_DEV))

        for h in range(N_DEV - 1):
            send_slot = h % 2
            recv_slot = (h + 1) % 2
            rdma = pltpu.make_async_remote_copy(
                src_ref=comm_ref.at[send_slot],
                dst_ref=comm_ref.at[recv_slot],
                send_sem=send_sems.at[send_slot],
                recv_sem=recv_sems.at[recv_slot],
                device_id=(right,),
                device_id_type=pl.DeviceIdType.MESH,
            )
            rdma.start()
            partial = chunk_partial(lax.rem(my + N_DEV - 2 - h, N_DEV))
            rdma.wait()
            if h < N_DEV - 2:
                comm_ref[recv_slot, :, :] = comm_ref[recv_slot, :, :] + partial
            else:
                y = comm_ref[recv_slot, :, :] + partial
                cg = 0.7978845608028654
                out_ref[:, :] = 0.5 * y * (
                    1.0 + jnp.tanh(cg * (y + 0.044715 * y * y * y))
                )

    return pl.pallas_call(
        body,
        out_shape=jax.ShapeDtypeStruct((m_per, n), jnp.float32),
        in_specs=[
            pl.BlockSpec(memory_space=pltpu.VMEM),
            pl.BlockSpec(memory_space=pltpu.VMEM),
        ],
        out_specs=pl.BlockSpec(memory_space=pltpu.VMEM),
        scratch_shapes=[
            pltpu.VMEM((2, m_per, n), jnp.float32),
            pltpu.SemaphoreType.DMA((2,)),
            pltpu.SemaphoreType.DMA((2,)),
        ],
        compiler_params=pltpu.CompilerParams(collective_id=0),
    )(x, w_mat)


# baseline (device time: 1414942 ns/iter reference)
import jax
import jax.numpy as jnp
from jax import lax
from jax.experimental import pallas as pl
from jax.experimental.pallas import tpu as pltpu

N_DEV = 16


def kernel(x, w_mat):
    m, k_per = x.shape
    _, n = w_mat.shape
    m_per = m // N_DEV

    def body(x_ref, w_ref, out_ref, comm_ref, send_sems, recv_sems):
        my = lax.axis_index("i")
        left = lax.rem(my + N_DEV - 1, N_DEV)
        right = lax.rem(my + 1, N_DEV)

        barrier_sem = pltpu.get_barrier_semaphore()
        for nbr in (left, right):
            pl.semaphore_signal(
                barrier_sem, inc=1,
                device_id=(nbr,), device_id_type=pl.DeviceIdType.MESH,
            )
        pl.semaphore_wait(barrier_sem, 2)

        w = w_ref[:, :]

        def chunk_partial(c):
            return jnp.dot(
                x_ref[pl.ds(c * m_per, m_per), :], w,
                preferred_element_type=jnp.float32,
            )

        comm_ref[0, :, :] = chunk_partial(lax.rem(my + N_DEV - 1, N_DEV))

        for h in range(N_DEV - 1):
            send_slot = h % 2
            recv_slot = (h + 1) % 2
            rdma = pltpu.make_async_remote_copy(
                src_ref=comm_ref.at[send_slot],
                dst_ref=comm_ref.at[recv_slot],
                send_sem=send_sems.at[send_slot],
                recv_sem=recv_sems.at[recv_slot],
                device_id=(right,),
                device_id_type=pl.DeviceIdType.MESH,
            )
            rdma.start()
            partial = chunk_partial(lax.rem(my + N_DEV - 2 - h, N_DEV))
            rdma.wait()
            if h < N_DEV - 2:
                comm_ref[recv_slot, :, :] = comm_ref[recv_slot, :, :] + partial
            else:
                y = comm_ref[recv_slot, :, :] + partial
                cg = 0.7978845608028654
                out_ref[:, :] = 0.5 * y * (
                    1.0 + jnp.tanh(cg * (y + 0.044715 * y * y * y))
                )

    return pl.pallas_call(
        body,
        out_shape=jax.ShapeDtypeStruct((m_per, n), jnp.float32),
        in_specs=[
            pl.BlockSpec(memory_space=pltpu.VMEM),
            pl.BlockSpec(memory_space=pltpu.VMEM),
        ],
        out_specs=pl.BlockSpec(memory_space=pltpu.VMEM),
        scratch_shapes=[
            pltpu.VMEM((2, m_per, n), jnp.float32),
            pltpu.SemaphoreType.DMA((2,)),
            pltpu.SemaphoreType.DMA((2,)),
        ],
        compiler_params=pltpu.CompilerParams(
            collective_id=0, vmem_limit_bytes=100 * 1024 * 1024
        ),
    )(x, w_mat)


# device time: 760544 ns/iter; 1.8604x vs baseline; 1.8604x over previous
import jax
import jax.numpy as jnp
from jax import lax
from jax.experimental import pallas as pl
from jax.experimental.pallas import tpu as pltpu

N_DEV = 16


def kernel(x, w_mat):
    m, k_per = x.shape
    _, n = w_mat.shape
    m_per = m // N_DEV
    nh = n // 2

    def body(x_ref, w_ref, out_ref, comm_r, comm_l,
             send_sems_r, recv_sems_r, send_sems_l, recv_sems_l):
        my = lax.axis_index("i")
        left = lax.rem(my + N_DEV - 1, N_DEV)
        right = lax.rem(my + 1, N_DEV)

        barrier_sem = pltpu.get_barrier_semaphore()
        for nbr in (left, right):
            pl.semaphore_signal(
                barrier_sem, inc=1,
                device_id=(nbr,), device_id_type=pl.DeviceIdType.MESH,
            )
        pl.semaphore_wait(barrier_sem, 2)

        def partial_r(c):
            return jnp.dot(
                x_ref[pl.ds(c * m_per, m_per), :], w_ref[:, :nh],
                preferred_element_type=jnp.float32,
            )

        def partial_l(c):
            return jnp.dot(
                x_ref[pl.ds(c * m_per, m_per), :], w_ref[:, nh:],
                preferred_element_type=jnp.float32,
            )

        def gelu(y):
            cg = 0.7978845608028654
            return 0.5 * y * (1.0 + jnp.tanh(cg * (y + 0.044715 * y * y * y)))

        comm_r[0, :, :] = partial_r(lax.rem(my + N_DEV - 1, N_DEV))
        comm_l[0, :, :] = partial_l(lax.rem(my + 1, N_DEV))

        for h in range(N_DEV - 1):
            ss = h % 2
            rs = (h + 1) % 2
            rdma_r = pltpu.make_async_remote_copy(
                src_ref=comm_r.at[ss], dst_ref=comm_r.at[rs],
                send_sem=send_sems_r.at[ss], recv_sem=recv_sems_r.at[rs],
                device_id=(right,), device_id_type=pl.DeviceIdType.MESH,
            )
            rdma_l = pltpu.make_async_remote_copy(
                src_ref=comm_l.at[ss], dst_ref=comm_l.at[rs],
                send_sem=send_sems_l.at[ss], recv_sem=recv_sems_l.at[rs],
                device_id=(left,), device_id_type=pl.DeviceIdType.MESH,
            )
            rdma_r.start()
            rdma_l.start()
            pr = partial_r(lax.rem(my + N_DEV - 2 - h, N_DEV))
            pl_ = partial_l(lax.rem(my + h + 2, N_DEV))
            rdma_r.wait()
            rdma_l.wait()
            if h < N_DEV - 2:
                comm_r[rs, :, :] = comm_r[rs, :, :] + pr
                comm_l[rs, :, :] = comm_l[rs, :, :] + pl_
            else:
                out_ref[:, :nh] = gelu(comm_r[rs, :, :] + pr)
                out_ref[:, nh:] = gelu(comm_l[rs, :, :] + pl_)

    return pl.pallas_call(
        body,
        out_shape=jax.ShapeDtypeStruct((m_per, n), jnp.float32),
        in_specs=[
            pl.BlockSpec(memory_space=pltpu.VMEM),
            pl.BlockSpec(memory_space=pltpu.VMEM),
        ],
        out_specs=pl.BlockSpec(memory_space=pltpu.VMEM),
        scratch_shapes=[
            pltpu.VMEM((2, m_per, nh), jnp.float32),
            pltpu.VMEM((2, m_per, nh), jnp.float32),
            pltpu.SemaphoreType.DMA((2,)),
            pltpu.SemaphoreType.DMA((2,)),
            pltpu.SemaphoreType.DMA((2,)),
            pltpu.SemaphoreType.DMA((2,)),
        ],
        compiler_params=pltpu.CompilerParams(
            collective_id=0, vmem_limit_bytes=100 * 1024 * 1024
        ),
    )(x, w_mat)


# device time: 701944 ns/iter; 2.0157x vs baseline; 1.0835x over previous
import jax
import jax.numpy as jnp
from jax import lax
from jax.experimental import pallas as pl
from jax.experimental.pallas import tpu as pltpu

N_DEV = 16
N_SUB = 2


def kernel(x, w_mat):
    m, k_per = x.shape
    _, n = w_mat.shape
    m_per = m // N_DEV
    nh = n // 2
    S = nh // N_SUB

    def body(x_ref, w_ref, out_ref, comm_r, comm_l,
             send_sems_r, recv_sems_r, send_sems_l, recv_sems_l):
        my = lax.axis_index("i")
        left = lax.rem(my + N_DEV - 1, N_DEV)
        right = lax.rem(my + 1, N_DEV)

        barrier_sem = pltpu.get_barrier_semaphore()
        for nbr in (left, right):
            pl.semaphore_signal(
                barrier_sem, inc=1,
                device_id=(nbr,), device_id_type=pl.DeviceIdType.MESH,
            )
        pl.semaphore_wait(barrier_sem, 2)

        def partial(c, c0, c1):
            return jnp.dot(
                x_ref[pl.ds(c * m_per, m_per), :], w_ref[:, c0:c1],
                preferred_element_type=jnp.float32,
            )

        def gelu(y):
            cg = 0.7978845608028654
            return 0.5 * y * (1.0 + jnp.tanh(cg * (y + 0.044715 * y * y * y)))

        rings = (
            (comm_r, send_sems_r, recv_sems_r, right, 0),
            (comm_l, send_sems_l, recv_sems_l, left, nh),
        )

        def chunk_idx(ring, h):
            if ring == 0:
                return lax.rem(my + N_DEV - 1 - h, N_DEV)
            return lax.rem(my + 1 + h, N_DEV)

        def make_desc(ring, h, sub):
            comm, s_sems, r_sems, tgt, _ = rings[ring]
            ss, rs = h % 2, (h + 1) % 2
            return pltpu.make_async_remote_copy(
                src_ref=comm.at[ss, :, sub * S:(sub + 1) * S],
                dst_ref=comm.at[rs, :, sub * S:(sub + 1) * S],
                send_sem=s_sems.at[ss, sub],
                recv_sem=r_sems.at[rs, sub],
                device_id=(tgt,), device_id_type=pl.DeviceIdType.MESH,
            )

        descs = {}

        def start_send(ring, h, sub):
            if h >= 2:
                descs[(ring, h - 2, sub)].wait_send()
            d = make_desc(ring, h, sub)
            descs[(ring, h, sub)] = d
            d.start()

        for sub in range(N_SUB):
            for ring in range(2):
                comm, _, _, _, base = rings[ring]
                comm[0, :, sub * S:(sub + 1) * S] = partial(
                    chunk_idx(ring, 0), base + sub * S, base + (sub + 1) * S
                )
                start_send(ring, 0, sub)

        for h in range(N_DEV - 1):
            rs = (h + 1) % 2
            p = [partial(chunk_idx(ring, h + 1), rings[ring][4],
                         rings[ring][4] + nh) for ring in range(2)]
            for sub in range(N_SUB):
                lo, hi = sub * S, (sub + 1) * S
                for ring in range(2):
                    comm, _, _, _, base = rings[ring]
                    descs[(ring, h, sub)].wait_recv()
                    if h < N_DEV - 2:
                        comm[rs, :, lo:hi] = comm[rs, :, lo:hi] + p[ring][:, lo:hi]
                        start_send(ring, h + 1, sub)
                    else:
                        out_ref[:, base + lo:base + hi] = gelu(
                            comm[rs, :, lo:hi] + p[ring][:, lo:hi]
                        )

        for sub in range(N_SUB):
            for ring in range(2):
                descs[(ring, N_DEV - 3, sub)].wait_send()
                descs[(ring, N_DEV - 2, sub)].wait_send()

    return pl.pallas_call(
        body,
        out_shape=jax.ShapeDtypeStruct((m_per, n), jnp.float32),
        in_specs=[
            pl.BlockSpec(memory_space=pltpu.VMEM),
            pl.BlockSpec(memory_space=pltpu.VMEM),
        ],
        out_specs=pl.BlockSpec(memory_space=pltpu.VMEM),
        scratch_shapes=[
            pltpu.VMEM((2, m_per, nh), jnp.float32),
            pltpu.VMEM((2, m_per, nh), jnp.float32),
            pltpu.SemaphoreType.DMA((2, N_SUB)),
            pltpu.SemaphoreType.DMA((2, N_SUB)),
            pltpu.SemaphoreType.DMA((2, N_SUB)),
            pltpu.SemaphoreType.DMA((2, N_SUB)),
        ],
        compiler_params=pltpu.CompilerParams(
            collective_id=0, vmem_limit_bytes=100 * 1024 * 1024
        ),
    )(x, w_mat)
